# Initial kernel scaffold; baseline (speedup 1.0000x reference)
import jax
import jax.numpy as jnp
from jax.experimental import pallas as pl


def kernel(x, palette):
    raise NotImplementedError("write your pallas kernel here")


# speedup vs baseline: 1.2767x; 1.2767x over previous
import jax
import jax.numpy as jnp
from jax.experimental import pallas as pl

_TILE = 1792


def _nn_kernel(x_ref, a_ref, b2_ref, p_ref, out_ref):
    x = x_ref[0]
    m = jax.lax.dot_general(
        x.astype(jnp.bfloat16), a_ref[...].astype(jnp.bfloat16),
        (((1,), (1,)), ((), ())),
        preferred_element_type=jnp.float32,
    )
    a2 = jnp.sum(x * x, axis=1, keepdims=True)
    sq = (a2 + m) + b2_ref[...]
    d = jnp.sqrt(jnp.maximum(sq, 0.0))
    dmin = jnp.min(d, axis=1, keepdims=True)
    iota = jax.lax.broadcasted_iota(jnp.int32, d.shape, 1)
    idx = jnp.min(jnp.where(d == dmin, iota, jnp.int32(1 << 30)), axis=1)
    onehot = (iota == idx[:, None]).astype(jnp.float32)
    out_ref[0] = jax.lax.dot(
        onehot, p_ref[...],
        precision=jax.lax.Precision.HIGHEST,
        preferred_element_type=jnp.float32,
    )


def kernel(x, palette):
    B, C, H, W = x.shape
    HW = H * W
    K = palette.shape[0]
    xt = x.reshape(B, C, HW).transpose(0, 2, 1)
    b2 = jnp.sum(palette * palette, axis=1)[None, :]
    a = -2.0 * palette
    nt = HW // _TILE
    out = pl.pallas_call(
        _nn_kernel,
        grid=(B, nt),
        in_specs=[
            pl.BlockSpec((1, _TILE, C), lambda b, t: (b, t, 0)),
            pl.BlockSpec((K, C), lambda b, t: (0, 0)),
            pl.BlockSpec((1, K), lambda b, t: (0, 0)),
            pl.BlockSpec((K, C), lambda b, t: (0, 0)),
        ],
        out_specs=pl.BlockSpec((1, _TILE, C), lambda b, t: (b, t, 0)),
        out_shape=jax.ShapeDtypeStruct((B, HW, C), x.dtype),
    )(xt, a, b2, palette)
    return out.transpose(0, 2, 1).reshape(B, C, H, W)
